# baseline (speedup 1.0000x reference)
import functools

import jax
import jax.numpy as jnp
from jax import lax
from jax.experimental import pallas as pl
from jax.experimental.pallas import tpu as pltpu
from jax.experimental.pallas import tpu_sc as plsc

NC = 2
NS = 16
NW = NC * NS
LANES = 16
SEQ_PER_BLK = 2
SLOTS = 4


def _build_table_prep(V, E):
    CB = 512
    grid_n = (V + CB - 1) // CB
    rows_per_step = CB * E // 128

    @functools.partial(
        pl.pallas_call,
        grid=(grid_n,),
        in_specs=[pl.BlockSpec((E, CB), lambda i: (0, i))],
        out_specs=pl.BlockSpec((rows_per_step, 128), lambda i: (i, 0)),
        out_shape=jax.ShapeDtypeStruct((V * E // 128, 128), jnp.float32),
    )
    def prep(tokT_ref, out_ref):
        x = tokT_ref[...]
        xt = x.T
        r = lax.broadcasted_iota(jnp.int32, (rows_per_step, CB), 0)
        c = lax.broadcasted_iota(jnp.int32, (rows_per_step, CB), 1)
        s_even = (c == 2 * r).astype(jnp.float32)
        s_odd = (c == 2 * r + 1).astype(jnp.float32)
        even = jax.lax.dot_general(
            s_even, xt, (((1,), (0,)), ((), ())),
            preferred_element_type=jnp.float32)
        odd = jax.lax.dot_general(
            s_odd, xt, (((1,), (0,)), ((), ())),
            preferred_element_type=jnp.float32)
        out_ref[...] = jnp.concatenate([even, odd], axis=1)

    return prep


def _build(B, L, E):
    assert B % NW == 0
    seqs_per_w = B // NW
    assert seqs_per_w % SEQ_PER_BLK == 0
    nblocks = seqs_per_w // SEQ_PER_BLK
    assert nblocks % SLOTS == 0
    nsteps = nblocks // SLOTS
    chunks = []
    off = 0
    while off < L:
        n = min(128, L - off)
        assert n % 8 == 0 and off % 8 == 0
        chunks.append((off, n))
        off += n

    mesh = plsc.VectorSubcoreMesh(
        core_axis_name="c", subcore_axis_name="s",
        num_cores=NC, num_subcores=NS)

    @functools.partial(
        pl.kernel,
        out_type=jax.ShapeDtypeStruct((B, L, E), jnp.float32),
        mesh=mesh,
        compiler_params=pltpu.CompilerParams(use_tc_tiling_on_sc=False),
        scratch_types=[
            pltpu.VMEM((SLOTS, SEQ_PER_BLK, L), jnp.int32),
            pltpu.VMEM((SLOTS, SEQ_PER_BLK, L, E), jnp.float32),
            pltpu.VMEM((L, E), jnp.float32),
            pltpu.SemaphoreType.DMA((SLOTS,)),
            pltpu.SemaphoreType.DMA((SLOTS,)),
            pltpu.SemaphoreType.DMA((SLOTS,)),
        ],
    )
    def emb_kernel(idx_hbm, tok_hbm, pos_hbm, out_hbm,
                   idx_v, rows_v, pos_v, isems, gsems, osems):
        wid = lax.axis_index("s") * NC + lax.axis_index("c")
        seq_base = wid * seqs_per_w

        pltpu.sync_copy(pos_hbm, pos_v)

        def idx_issue(j, t):
            pltpu.async_copy(
                idx_hbm.at[pl.ds(seq_base + j * SEQ_PER_BLK, SEQ_PER_BLK)],
                idx_v.at[t], isems.at[t])

        def idx_wait(t):
            pltpu.make_async_copy(idx_hbm.at[pl.ds(0, SEQ_PER_BLK)],
                                  idx_v.at[t], isems.at[t]).wait()

        def gather_issue(s):
            for q in range(SEQ_PER_BLK):
                for off, n in chunks:
                    pltpu.async_copy(
                        tok_hbm.at[idx_v.at[s, q, pl.ds(off, n)]],
                        rows_v.at[s, q, pl.ds(off, n)],
                        gsems.at[s])

        def gather_wait(s):
            pltpu.make_async_copy(out_hbm.at[pl.ds(0, SEQ_PER_BLK)],
                                  rows_v.at[s], gsems.at[s]).wait()

        def scatter_issue(j, s):
            pltpu.async_copy(
                rows_v.at[s],
                out_hbm.at[pl.ds(seq_base + j * SEQ_PER_BLK, SEQ_PER_BLK)],
                osems.at[s])

        def scatter_wait(s):
            pltpu.make_async_copy(rows_v.at[s],
                                  out_hbm.at[pl.ds(0, SEQ_PER_BLK)],
                                  osems.at[s]).wait()

        def add_pos(s):
            def body(l, c):
                for k in range(E // LANES):
                    pv = pos_v[l, pl.ds(k * LANES, LANES)]
                    for q in range(SEQ_PER_BLK):
                        plsc.addupdate(
                            rows_v.at[s, q, l, pl.ds(k * LANES, LANES)], pv)
                return c
            lax.fori_loop(0, L, body, 0)

        pltpu.sync_copy(idx_hbm.at[pl.ds(seq_base, SEQ_PER_BLK)], idx_v.at[0])
        gather_issue(0)
        idx_issue(1, 1)

        def step_body(g, carry):
            for s in range(SLOTS):
                j = SLOTS * g + s

                def do_step():
                    idx_wait(s)
                    gather_issue(s)

                def do_osem_wait():
                    scatter_wait(s)

                def do_idx_issue():
                    idx_issue(j + 1, (s + 1) % SLOTS)

                def do_compute():
                    sp = (s - 1) % SLOTS
                    gather_wait(sp)
                    add_pos(sp)
                    scatter_issue(j - 1, sp)

                if s == 0:
                    pl.when(g > 0)(do_osem_wait)
                    pl.when(g > 0)(do_step)
                    pl.when(g > 0)(do_idx_issue)
                    pl.when(g > 0)(do_compute)
                else:
                    pl.when(g > 0)(do_osem_wait)
                    do_step()
                    if s == SLOTS - 1:
                        pl.when(g < nsteps - 1)(do_idx_issue)
                    else:
                        do_idx_issue()
                    do_compute()
            return carry

        lax.fori_loop(0, nsteps, step_body, 0)

        last = SLOTS - 1
        gather_wait(last)
        add_pos(last)
        scatter_issue(nblocks - 1, last)
        for s in range(SLOTS):
            scatter_wait(s)

    return emb_kernel


@jax.jit
def kernel(inputs, token_table, pos_table):
    B, L = inputs.shape
    V, E = token_table.shape
    tok_lin = _build_table_prep(V, E)(token_table.T).reshape(V, E)
    emb = _build(B, L, E)
    return emb(inputs.astype(jnp.int32), tok_lin, pos_table)


# speedup vs baseline: 1.1141x; 1.1141x over previous
import functools

import jax
import jax.numpy as jnp
from jax import lax
from jax.experimental import pallas as pl
from jax.experimental.pallas import tpu as pltpu
from jax.experimental.pallas import tpu_sc as plsc

NC = 2
NS = 16
NW = NC * NS
LANES = 16
SLOTS = 4
LCHUNK = 100


def _build(B, L, E, V):
    assert B == 128 * NW
    assert E % 8 == 0 and E % LANES == 0
    NA = E // 8
    assert L % LCHUNK == 0 and LCHUNK % SLOTS == 0
    nhalves = L // LCHUNK
    nsteps = LCHUNK // SLOTS

    mesh = plsc.VectorSubcoreMesh(
        core_axis_name="c", subcore_axis_name="s",
        num_cores=NC, num_subcores=NS)

    @functools.partial(
        pl.kernel,
        out_type=jax.ShapeDtypeStruct((L, NA, B // 128, 8, 128), jnp.float32),
        mesh=mesh,
        compiler_params=pltpu.CompilerParams(
            use_tc_tiling_on_sc=False, needs_layout_passes=False),
        scratch_types=[
            pltpu.VMEM((LCHUNK, 128), jnp.int32),
            pltpu.VMEM((SLOTS, 128), jnp.int32),
            pltpu.VMEM((SLOTS, 128, 128), jnp.float32),
            pltpu.VMEM((SLOTS, NA * 8, 128), jnp.float32),
            pltpu.VMEM((L, E), jnp.float32),
            pltpu.SemaphoreType.DMA((SLOTS,)),
            pltpu.SemaphoreType.DMA((SLOTS,)),
        ],
    )
    def emb_kernel(idx_hbm, tok_hbm, pos_hbm, out_hbm,
                   idx_v, half_v, rows_v, trans_v, pos_v, gsems, osems):
        wid = lax.axis_index("s") * NC + lax.axis_index("c")

        pltpu.sync_copy(pos_hbm, pos_v)

        e_iota = lax.iota(jnp.int32, LANES)

        def gather_issue(s):
            pltpu.async_copy(tok_hbm.at[half_v.at[s]], rows_v.at[s],
                             gsems.at[s])

        def gather_wait(s):
            pltpu.make_async_copy(tok_hbm.at[pl.ds(0, 128)],
                                  rows_v.at[s], gsems.at[s]).wait()

        def halve_idx(ll, s):
            for i in range(128 // LANES):
                half_v[s, pl.ds(i * LANES, LANES)] = (
                    idx_v[ll, pl.ds(i * LANES, LANES)] >> 1)

        def scatter_issue(l, s):
            for a in range(NA):
                pltpu.async_copy(trans_v.at[s, pl.ds(a * 8, 8)],
                                 out_hbm.at[l, a, wid],
                                 osems.at[s])

        def scatter_wait(s):
            for a in range(NA):
                pltpu.make_async_copy(trans_v.at[s, pl.ds(a * 8, 8)],
                                      out_hbm.at[0, a, 0],
                                      osems.at[s]).wait()

        e_vecs = [e_iota + (k * LANES) for k in range(E // LANES)]

        def compute(l, ll, s):
            pvs = [pos_v[l, pl.ds(k * LANES, LANES)]
                   for k in range(E // LANES)]

            def body(gb, carry):
                iv = idx_v[ll, pl.ds(gb * LANES, LANES)]
                par = iv & 1
                for i in range(LANES):
                    b = gb * LANES + i
                    off = par[i] * E
                    m_vec = jnp.full((LANES,), b, dtype=jnp.int32)
                    for k in range(E // LANES):
                        val = (rows_v[s, b, pl.ds(off + k * LANES, LANES)]
                               + pvs[k])
                        plsc.store_scatter(trans_v.at[s],
                                           [e_vecs[k], m_vec], val)
                return carry
            lax.fori_loop(0, 128 // LANES, body, 0)

        def run_half(h):
            lbase = h * LCHUNK
            pltpu.sync_copy(
                idx_hbm.at[pl.ds(lbase, LCHUNK), pl.ds(wid * 128, 128)],
                idx_v)

            halve_idx(0, 0)
            gather_issue(0)

            def step_body(g, carry):
                for s in range(SLOTS):
                    ll = SLOTS * g + s
                    sp = (s - 1) % SLOTS

                    def do_gather():
                        halve_idx(ll, s)
                        gather_issue(s)

                    def do_osem_wait():
                        scatter_wait(sp)

                    def do_compute():
                        gather_wait(sp)
                        compute(lbase + ll - 1, ll - 1, sp)
                        scatter_issue(lbase + ll - 1, sp)

                    if s == 0:
                        pl.when(g > 0)(do_gather)
                        if h > 0:
                            pl.when(g > 0)(do_osem_wait)
                        else:
                            pl.when(g > 1)(do_osem_wait)
                        pl.when(g > 0)(do_compute)
                    else:
                        do_gather()
                        if h > 0:
                            do_osem_wait()
                        else:
                            pl.when(g > 0)(do_osem_wait)
                        do_compute()
                return carry

            lax.fori_loop(0, nsteps, step_body, 0)

            last = SLOTS - 1
            gather_wait(last)
            scatter_wait(last)
            compute(lbase + LCHUNK - 1, LCHUNK - 1, last)
            scatter_issue(lbase + LCHUNK - 1, last)

        for h in range(nhalves):
            run_half(h)
        for s in range(SLOTS):
            scatter_wait(s)

    return emb_kernel


@jax.jit
def kernel(inputs, token_table, pos_table):
    B, L = inputs.shape
    V, E = token_table.shape
    idxT = inputs.astype(jnp.int32).T
    tok2 = token_table.reshape(V // 2, 2 * E)
    out5 = _build(B, L, E, V)(idxT, tok2, pos_table)
    return out5.transpose(2, 4, 0, 1, 3).reshape(B, L, E)


# speedup vs baseline: 1.1370x; 1.0205x over previous
import functools

import jax
import jax.numpy as jnp
from jax import lax
from jax.experimental import pallas as pl
from jax.experimental.pallas import tpu as pltpu
from jax.experimental.pallas import tpu_sc as plsc

NC = 2
NS = 16
NW = NC * NS
LANES = 16
SLOTS = 4
LCHUNK = 100


def _build(B, L, E, V):
    assert B == 128 * NW
    assert E % 8 == 0 and E % LANES == 0
    NA = E // 8
    assert L % LCHUNK == 0 and LCHUNK % SLOTS == 0
    nhalves = L // LCHUNK
    nsteps = LCHUNK // SLOTS

    mesh = plsc.VectorSubcoreMesh(
        core_axis_name="c", subcore_axis_name="s",
        num_cores=NC, num_subcores=NS)

    @functools.partial(
        pl.kernel,
        out_type=jax.ShapeDtypeStruct((L, NA, B // 128, 8, 128), jnp.float32),
        mesh=mesh,
        compiler_params=pltpu.CompilerParams(
            use_tc_tiling_on_sc=False, needs_layout_passes=False),
        scratch_types=[
            pltpu.VMEM((LCHUNK, 128), jnp.int32),
            pltpu.VMEM((SLOTS, 128, 64), jnp.float32),
            pltpu.VMEM((SLOTS, NA * 8, 128), jnp.float32),
            pltpu.VMEM((L, E), jnp.float32),
            pltpu.SemaphoreType.DMA((SLOTS,)),
            pltpu.SemaphoreType.DMA((SLOTS,)),
        ],
    )
    def emb_kernel(idx_hbm, tok_hbm, pos_hbm, out_hbm,
                   idx_v, rows_v, trans_v, pos_v, gsems, osems):
        wid = lax.axis_index("s") * NC + lax.axis_index("c")

        pltpu.sync_copy(pos_hbm, pos_v)

        e_iota = lax.iota(jnp.int32, LANES)

        def gather_issue(ll, s):
            pltpu.async_copy(tok_hbm.at[idx_v.at[ll]], rows_v.at[s],
                             gsems.at[s])

        def gather_wait(s):
            pltpu.make_async_copy(tok_hbm.at[pl.ds(0, 128)],
                                  rows_v.at[s], gsems.at[s]).wait()

        def scatter_issue(l, s):
            for a in range(NA):
                pltpu.async_copy(trans_v.at[s, pl.ds(a * 8, 8)],
                                 out_hbm.at[l, a, wid],
                                 osems.at[s])

        def scatter_wait(s):
            for a in range(NA):
                pltpu.make_async_copy(trans_v.at[s, pl.ds(a * 8, 8)],
                                      out_hbm.at[0, a, 0],
                                      osems.at[s]).wait()

        e_vecs = [e_iota + (k * LANES) for k in range(E // LANES)]

        def compute(l, ll, s):
            pvs = [pos_v[l, pl.ds(k * LANES, LANES)]
                   for k in range(E // LANES)]

            def body(gb, carry):
                for i in range(LANES):
                    b = gb * LANES + i
                    m_vec = jnp.full((LANES,), b, dtype=jnp.int32)
                    for k in range(E // LANES):
                        val = (rows_v[s, b, pl.ds(k * LANES, LANES)]
                               + pvs[k])
                        plsc.store_scatter(trans_v.at[s],
                                           [e_vecs[k], m_vec], val)
                return carry
            lax.fori_loop(0, 128 // LANES, body, 0)

        def run_half(h):
            lbase = h * LCHUNK
            pltpu.sync_copy(
                idx_hbm.at[pl.ds(lbase, LCHUNK), pl.ds(wid * 128, 128)],
                idx_v)

            gather_issue(0, 0)

            def step_body(g, carry):
                for s in range(SLOTS):
                    ll = SLOTS * g + s
                    sp = (s - 1) % SLOTS

                    def do_gather():
                        gather_issue(ll, s)

                    def do_osem_wait():
                        scatter_wait(sp)

                    def do_compute():
                        gather_wait(sp)
                        compute(lbase + ll - 1, ll - 1, sp)
                        scatter_issue(lbase + ll - 1, sp)

                    if s == 0:
                        pl.when(g > 0)(do_gather)
                        if h > 0:
                            pl.when(g > 0)(do_osem_wait)
                        else:
                            pl.when(g > 1)(do_osem_wait)
                        pl.when(g > 0)(do_compute)
                    else:
                        do_gather()
                        if h > 0:
                            do_osem_wait()
                        else:
                            pl.when(g > 0)(do_osem_wait)
                        do_compute()
                return carry

            lax.fori_loop(0, nsteps, step_body, 0)

            last = SLOTS - 1
            gather_wait(last)
            scatter_wait(last)
            compute(lbase + LCHUNK - 1, LCHUNK - 1, last)
            scatter_issue(lbase + LCHUNK - 1, last)

        for h in range(nhalves):
            run_half(h)
        for s in range(SLOTS):
            scatter_wait(s)

    return emb_kernel


@jax.jit
def kernel(inputs, token_table, pos_table):
    B, L = inputs.shape
    V, E = token_table.shape
    idxT = inputs.astype(jnp.int32).T
    out5 = _build(B, L, E, V)(idxT, token_table, pos_table)
    return out5.transpose(2, 4, 0, 1, 3).reshape(B, L, E)


# speedup vs baseline: 1.3993x; 1.2307x over previous
import functools

import jax
import jax.numpy as jnp
from jax import lax
from jax.experimental import pallas as pl
from jax.experimental.pallas import tpu as pltpu
from jax.experimental.pallas import tpu_sc as plsc

NC = 2
NS = 16
NW = NC * NS
LANES = 16
SLOTS = 2
LCHUNK = 100


def _build(B, L, E, V):
    assert B == 128 * NW
    assert E % 8 == 0 and E % LANES == 0
    NA = E // 8
    assert L % LCHUNK == 0 and LCHUNK % SLOTS == 0
    nhalves = L // LCHUNK
    nsteps = LCHUNK // SLOTS

    mesh = plsc.VectorSubcoreMesh(
        core_axis_name="c", subcore_axis_name="s",
        num_cores=NC, num_subcores=NS)

    @functools.partial(
        pl.kernel,
        out_type=jax.ShapeDtypeStruct((L, NA, B // 128, 8, 128), jnp.float32),
        mesh=mesh,
        compiler_params=pltpu.CompilerParams(
            use_tc_tiling_on_sc=False, needs_layout_passes=False),
        scratch_types=[
            pltpu.VMEM((LCHUNK, 128), jnp.int32),
            pltpu.VMEM((SLOTS, 128, 64), jnp.float32),
            pltpu.VMEM((SLOTS, 16, 128), jnp.float32),
            pltpu.VMEM((SLOTS, 16, 128), jnp.float32),
            pltpu.VMEM((SLOTS, 16, 128), jnp.float32),
            pltpu.VMEM((SLOTS, 16, 128), jnp.float32),
            pltpu.VMEM((L, E), jnp.float32),
            pltpu.SemaphoreType.DMA((SLOTS,)),
            pltpu.SemaphoreType.DMA((SLOTS,)),
        ],
    )
    def emb_kernel(idx_hbm, tok_hbm, pos_hbm, out_hbm,
                   idx_v, rows_v, trans_0, trans_1, trans_2, trans_3,
                   pos_v, gsems, osems):
        trans = (trans_0, trans_1, trans_2, trans_3)
        wid = lax.axis_index("s") * NC + lax.axis_index("c")

        pltpu.sync_copy(pos_hbm, pos_v)

        e_iota = lax.iota(jnp.int32, LANES)

        def gather_issue(ll, s):
            pltpu.async_copy(tok_hbm.at[idx_v.at[ll]], rows_v.at[s],
                             gsems.at[s])

        def gather_wait(s):
            pltpu.make_async_copy(tok_hbm.at[pl.ds(0, 128)],
                                  rows_v.at[s], gsems.at[s]).wait()

        def scatter_issue(l, s):
            for a in range(NA):
                pltpu.async_copy(trans[a // 2].at[s, pl.ds((a % 2) * 8, 8)],
                                 out_hbm.at[l, a, wid],
                                 osems.at[s])

        def scatter_wait(s):
            for a in range(NA):
                pltpu.make_async_copy(
                    trans[a // 2].at[s, pl.ds((a % 2) * 8, 8)],
                    out_hbm.at[0, a, 0],
                    osems.at[s]).wait()

        def compute(l, ll, s):
            pvs = [pos_v[l, pl.ds(k * LANES, LANES)]
                   for k in range(E // LANES)]

            @plsc.parallel_loop(0, 128, step=LANES)
            def body(b0):
                for i in range(LANES):
                    b = b0 + i
                    m_vec = jnp.full((LANES,), b, dtype=jnp.int32)
                    for k in range(E // LANES):
                        val = (rows_v[s, b, pl.ds(k * LANES, LANES)]
                               + pvs[k])
                        plsc.store_scatter(trans[k].at[s],
                                           [e_iota, m_vec], val)

        def run_half(h):
            lbase = h * LCHUNK
            pltpu.sync_copy(
                idx_hbm.at[pl.ds(lbase, LCHUNK), pl.ds(wid * 128, 128)],
                idx_v)

            gather_issue(0, 0)

            def step_body(g, carry):
                for s in range(SLOTS):
                    ll = SLOTS * g + s
                    sp = (s - 1) % SLOTS

                    def do_gather():
                        gather_issue(ll, s)

                    def do_osem_wait():
                        scatter_wait(sp)

                    def do_compute():
                        gather_wait(sp)
                        compute(lbase + ll - 1, ll - 1, sp)
                        scatter_issue(lbase + ll - 1, sp)

                    if s == 0:
                        pl.when(g > 0)(do_gather)
                        if h > 0:
                            pl.when(g > 0)(do_osem_wait)
                        else:
                            pl.when(g > 1)(do_osem_wait)
                        pl.when(g > 0)(do_compute)
                    else:
                        do_gather()
                        if h > 0:
                            do_osem_wait()
                        else:
                            pl.when(g > 0)(do_osem_wait)
                        do_compute()
                return carry

            lax.fori_loop(0, nsteps, step_body, 0)

            last = SLOTS - 1
            gather_wait(last)
            scatter_wait(last)
            compute(lbase + LCHUNK - 1, LCHUNK - 1, last)
            scatter_issue(lbase + LCHUNK - 1, last)

        for h in range(nhalves):
            run_half(h)
        for s in range(SLOTS):
            scatter_wait(s)

    return emb_kernel


@jax.jit
def kernel(inputs, token_table, pos_table):
    B, L = inputs.shape
    V, E = token_table.shape
    idxT = inputs.astype(jnp.int32).T
    out5 = _build(B, L, E, V)(idxT, token_table, pos_table)
    return out5.transpose(2, 4, 0, 1, 3).reshape(B, L, E)


# speedup vs baseline: 2.1010x; 1.5015x over previous
import functools

import jax
import jax.numpy as jnp
from jax import lax
from jax.experimental import pallas as pl
from jax.experimental.pallas import tpu as pltpu
from jax.experimental.pallas import tpu_sc as plsc

NC = 2
NS = 16
NW = NC * NS
LANES = 16
SEQ_PER_BLK = 2
SLOTS = 4


def _build(B, L, E):
    assert B % NW == 0
    seqs_per_w = B // NW
    assert seqs_per_w % SEQ_PER_BLK == 0
    nblocks = seqs_per_w // SEQ_PER_BLK
    assert nblocks % SLOTS == 0
    nsteps = nblocks // SLOTS
    chunks = []
    off = 0
    while off < L:
        n = min(128, L - off)
        assert n % 8 == 0 and off % 8 == 0
        chunks.append((off, n))
        off += n

    mesh = plsc.VectorSubcoreMesh(
        core_axis_name="c", subcore_axis_name="s",
        num_cores=NC, num_subcores=NS)

    @functools.partial(
        pl.kernel,
        out_type=jax.ShapeDtypeStruct((B, L, 2 * E), jnp.float32),
        mesh=mesh,
        compiler_params=pltpu.CompilerParams(use_tc_tiling_on_sc=False),
        scratch_types=[
            pltpu.VMEM((SLOTS, SEQ_PER_BLK, L), jnp.int32),
            pltpu.VMEM((SLOTS, SEQ_PER_BLK, L, E), jnp.float32),
            pltpu.VMEM((L, E), jnp.float32),
            pltpu.SemaphoreType.DMA((SLOTS,)),
            pltpu.SemaphoreType.DMA((SLOTS,)),
            pltpu.SemaphoreType.DMA((SLOTS,)),
        ],
    )
    def emb_kernel(idx_hbm, tok_hbm, pos_hbm, out_hbm,
                   idx_v, rows_v, pos_v, isems, gsems, osems):
        wid = lax.axis_index("s") * NC + lax.axis_index("c")
        seq_base = wid * seqs_per_w

        pltpu.sync_copy(pos_hbm, pos_v)

        def idx_issue(j, t):
            pltpu.async_copy(
                idx_hbm.at[pl.ds(seq_base + j * SEQ_PER_BLK, SEQ_PER_BLK)],
                idx_v.at[t], isems.at[t])

        def idx_wait(t):
            pltpu.make_async_copy(idx_hbm.at[pl.ds(0, SEQ_PER_BLK)],
                                  idx_v.at[t], isems.at[t]).wait()

        def gather_issue(s):
            for q in range(SEQ_PER_BLK):
                for off, n in chunks:
                    pltpu.async_copy(
                        tok_hbm.at[idx_v.at[s, q, pl.ds(off, n)]],
                        rows_v.at[s, q, pl.ds(off, n)],
                        gsems.at[s])

        def gather_wait(s):
            pltpu.make_async_copy(out_hbm.at[pl.ds(0, SEQ_PER_BLK)],
                                  rows_v.at[s], gsems.at[s]).wait()

        def scatter_issue(j, s):
            pltpu.async_copy(
                rows_v.at[s],
                out_hbm.at[pl.ds(seq_base + j * SEQ_PER_BLK, SEQ_PER_BLK),
                           pl.ds(0, L), pl.ds(0, E)],
                osems.at[s])

        def scatter_wait(s):
            pltpu.make_async_copy(
                rows_v.at[s],
                out_hbm.at[pl.ds(0, SEQ_PER_BLK), pl.ds(0, L), pl.ds(0, E)],
                osems.at[s]).wait()

        def add_pos(s):
            def body(l, c):
                for k in range(E // LANES):
                    pv = pos_v[l, pl.ds(k * LANES, LANES)]
                    for q in range(SEQ_PER_BLK):
                        plsc.addupdate(
                            rows_v.at[s, q, l, pl.ds(k * LANES, LANES)], pv)
                return c
            lax.fori_loop(0, L, body, 0)

        pltpu.sync_copy(idx_hbm.at[pl.ds(seq_base, SEQ_PER_BLK)], idx_v.at[0])
        gather_issue(0)
        idx_issue(1, 1)

        def step_body(g, carry):
            for s in range(SLOTS):
                j = SLOTS * g + s

                def do_step():
                    idx_wait(s)
                    gather_issue(s)

                def do_osem_wait():
                    scatter_wait(s)

                def do_idx_issue():
                    idx_issue(j + 1, (s + 1) % SLOTS)

                def do_compute():
                    sp = (s - 1) % SLOTS
                    gather_wait(sp)
                    add_pos(sp)
                    scatter_issue(j - 1, sp)

                if s == 0:
                    pl.when(g > 0)(do_osem_wait)
                    pl.when(g > 0)(do_step)
                    pl.when(g > 0)(do_idx_issue)
                    pl.when(g > 0)(do_compute)
                else:
                    pl.when(g > 0)(do_osem_wait)
                    do_step()
                    if s == SLOTS - 1:
                        pl.when(g < nsteps - 1)(do_idx_issue)
                    else:
                        do_idx_issue()
                    do_compute()
            return carry

        lax.fori_loop(0, nsteps, step_body, 0)

        last = SLOTS - 1
        gather_wait(last)
        add_pos(last)
        scatter_issue(nblocks - 1, last)
        for s in range(SLOTS):
            scatter_wait(s)

    return emb_kernel


@jax.jit
def kernel(inputs, token_table, pos_table):
    B, L = inputs.shape
    E = token_table.shape[1]
    emb = _build(B, L, E)
    out_pad = emb(inputs.astype(jnp.int32), token_table, pos_table)
    return lax.slice(out_pad, (0, 0, 0), (B, L, E))
